# Initial kernel scaffold; baseline (speedup 1.0000x reference)
import functools

import jax
import jax.numpy as jnp
from jax import lax
from jax.experimental import pallas as pl
from jax.experimental.pallas import tpu as pltpu
from jax.experimental.pallas import tpu_sc as plsc

N = 50000
D = 512
B = 64
NW = 32
PER = 1568
S = 128
NEG = jnp.float32(-3.0e38)

_mesh = plsc.VectorSubcoreMesh(core_axis_name="c", subcore_axis_name="s")


def _insert3(cs, ci, t0s, t1s, t2s, t0i, t1i, t2i):
    gt0 = cs > t0s
    gt1 = cs > t1s
    gt2 = cs > t2s
    n0s = jnp.where(gt0, cs, t0s)
    n0i = jnp.where(gt0, ci, t0i)
    n1s = jnp.where(gt0, t0s, jnp.where(gt1, cs, t1s))
    n1i = jnp.where(gt0, t0i, jnp.where(gt1, ci, t1i))
    n2s = jnp.where(gt1, t1s, jnp.where(gt2, cs, t2s))
    n2i = jnp.where(gt1, t1i, jnp.where(gt2, ci, t2i))
    return n0s, n1s, n2s, n0i, n1i, n2i


@functools.partial(
    pl.kernel,
    out_type=(
        jax.ShapeDtypeStruct((2, B + 1, D), jnp.float32),
        jax.ShapeDtypeStruct((2, B + 1, 16), jnp.float32),
        jax.ShapeDtypeStruct((NW, B * 3), jnp.float32),
        jax.ShapeDtypeStruct((NW, B * 3), jnp.int32),
    ),
    mesh=_mesh,
    scratch_types=[
        pltpu.VMEM((S, D), jnp.float32),
        pltpu.VMEM((S,), jnp.int32),
        pltpu.VMEM((S, 16), jnp.float32),
        pltpu.VMEM((B * 3,), jnp.float32),
        pltpu.VMEM((B * 3,), jnp.int32),
        pltpu.VMEM_SHARED((B + 1, D), jnp.float32),
        pltpu.VMEM_SHARED((B + 1, 16), jnp.float32),
    ],
)
def _k1(x_hbm, bat_hbm, zsum_hbm, zcnt_hbm,
        psum_hbm, pcnt_hbm, cands_hbm, candi_hbm,
        rows_v, idx_v, ones_v, ts_s, ts_i, acc_sum, acc_cnt):
    cid = lax.axis_index("c")
    sid = lax.axis_index("s")
    wid = cid * 16 + sid
    start = wid * PER
    rows = jnp.minimum(PER, N - start)
    nc = (rows + (S - 1)) // S

    @pl.when(sid == 0)
    def _():
        pltpu.sync_copy(zsum_hbm, acc_sum)
        pltpu.sync_copy(zcnt_hbm, acc_cnt)

    def init_body(k, _):
        ts_s[pl.ds(k * 16, 16)] = jnp.full((16,), NEG, jnp.float32)
        ts_i[pl.ds(k * 16, 16)] = jnp.zeros((16,), jnp.int32)
        return 0

    lax.fori_loop(0, (B * 3) // 16, init_body, 0)
    plsc.subcore_barrier()

    def chunk_body(j, _):
        cstart = jnp.minimum(start + j * S, start + rows - S)
        fresh_from = start + j * S
        pltpu.sync_copy(x_hbm.at[pl.ds(cstart, S)], rows_v)
        pltpu.sync_copy(bat_hbm.at[pl.ds(cstart, S)], idx_v)

        def row_body(i, _):
            g = cstart + i
            fresh = g >= fresh_from
            b = idx_v[i]
            idx_v[i] = jnp.where(fresh, b, B)
            one = jnp.where(fresh, jnp.float32(1.0), jnp.float32(0.0))
            ones_v[i, :] = jnp.broadcast_to(one, (16,))
            sc = rows_v[i, D - 1]
            se = jnp.where(fresh, sc, NEG)
            base3 = b * 3
            t0s = ts_s[base3]
            t1s = ts_s[base3 + 1]
            t2s = ts_s[base3 + 2]
            t0i = ts_i[base3]
            t1i = ts_i[base3 + 1]
            t2i = ts_i[base3 + 2]
            n0s, n1s, n2s, n0i, n1i, n2i = _insert3(
                se, g, t0s, t1s, t2s, t0i, t1i, t2i)
            ts_s[base3] = n0s
            ts_s[base3 + 1] = n1s
            ts_s[base3 + 2] = n2s
            ts_i[base3] = n0i
            ts_i[base3 + 1] = n1i
            ts_i[base3 + 2] = n2i
            return 0

        lax.fori_loop(0, S, row_body, 0)
        pltpu.sync_copy(rows_v, acc_sum.at[idx_v], add=True)
        pltpu.sync_copy(ones_v, acc_cnt.at[idx_v], add=True)
        return 0

    lax.fori_loop(0, nc, chunk_body, 0)
    plsc.subcore_barrier()

    @pl.when(sid == 0)
    def _():
        pltpu.sync_copy(acc_sum, psum_hbm.at[cid])
        pltpu.sync_copy(acc_cnt, pcnt_hbm.at[cid])

    pltpu.sync_copy(ts_s, cands_hbm.at[wid])
    pltpu.sync_copy(ts_i, candi_hbm.at[wid])


@functools.partial(
    pl.kernel,
    out_type=jax.ShapeDtypeStruct((B, 5 * D), jnp.float32),
    mesh=_mesh,
    scratch_types=[
        pltpu.VMEM((NW, B * 3), jnp.float32),
        pltpu.VMEM((NW, B * 3), jnp.int32),
        pltpu.VMEM((2, B + 1, 16), jnp.float32),
        pltpu.VMEM((D,), jnp.float32),
        pltpu.VMEM((D,), jnp.float32),
        pltpu.VMEM((8,), jnp.int32),
        pltpu.VMEM((8, D), jnp.float32),
        pltpu.VMEM((5 * D,), jnp.float32),
        pltpu.SemaphoreType.DMA,
    ],
)
def _k2(x_hbm, psum_hbm, pcnt_hbm, cands_hbm, candi_hbm, out_hbm,
        cs_v, ci_v, pc_v, ps0_v, ps1_v, gi_v, grows_v, orow_v, sem):
    cid = lax.axis_index("c")
    sid = lax.axis_index("s")
    wid = cid * 16 + sid
    pltpu.sync_copy(cands_hbm, cs_v)
    pltpu.sync_copy(candi_hbm, ci_v)
    pltpu.sync_copy(pcnt_hbm, pc_v)

    def do_seg(seg):
        def m_body(t, carry):
            t0s, t1s, t2s, t0i, t1i, t2i = carry
            for k in range(3):
                cs = cs_v[t, seg * 3 + k]
                ci = ci_v[t, seg * 3 + k]
                t0s, t1s, t2s, t0i, t1i, t2i = _insert3(
                    cs, ci, t0s, t1s, t2s, t0i, t1i, t2i)
            return (t0s, t1s, t2s, t0i, t1i, t2i)

        z = jnp.int32(0)
        t0s, t1s, t2s, t0i, t1i, t2i = lax.fori_loop(
            0, NW, m_body, (NEG, NEG, NEG, z, z, z))

        cnt = pc_v[0, seg, :] + pc_v[1, seg, :]
        cntc = jnp.maximum(cnt, jnp.float32(1.0))
        one = jnp.full((16,), 1.0, jnp.float32)
        zero = jnp.zeros((16,), jnp.float32)
        v0 = jnp.where(cnt > 0.5, one, zero)
        v1 = jnp.where(cnt > 1.5, one, zero)
        v2 = jnp.where(cnt > 2.5, one, zero)

        gi_v[0] = t0i
        gi_v[1] = t1i
        gi_v[2] = t2i
        for k in range(3, 8):
            gi_v[k] = jnp.int32(0)
        pltpu.async_copy(x_hbm.at[gi_v], grows_v, sem).wait()
        pltpu.sync_copy(psum_hbm.at[0, seg], ps0_v)
        pltpu.sync_copy(psum_hbm.at[1, seg], ps1_v)

        def col_body(ccol, _):
            sl = pl.ds(ccol * 16, 16)
            sv = ps0_v[sl] + ps1_v[sl]
            orow_v[pl.ds(ccol * 16, 16)] = sv / cntc
            orow_v[pl.ds(D + ccol * 16, 16)] = sv
            orow_v[pl.ds(2 * D + ccol * 16, 16)] = grows_v[0, sl] * v0
            orow_v[pl.ds(3 * D + ccol * 16, 16)] = grows_v[1, sl] * v1
            orow_v[pl.ds(4 * D + ccol * 16, 16)] = grows_v[2, sl] * v2
            return 0

        lax.fori_loop(0, D // 16, col_body, 0)
        pltpu.sync_copy(orow_v, out_hbm.at[seg])

    do_seg(wid * 2)
    do_seg(wid * 2 + 1)


def kernel(x, batch):
    bat = batch.astype(jnp.int32)
    zsum = jnp.zeros((B + 1, D), jnp.float32)
    zcnt = jnp.zeros((B + 1, 16), jnp.float32)
    psum, pcnt, cs, ci = _k1(x, bat, zsum, zcnt)
    return _k2(x, psum, pcnt, cs, ci)


# speedup vs baseline: 3.3597x; 3.3597x over previous
import functools

import jax
import jax.numpy as jnp
from jax import lax
from jax.experimental import pallas as pl
from jax.experimental.pallas import tpu as pltpu
from jax.experimental.pallas import tpu_sc as plsc

N = 50000
D = 512
B = 64
NW = 32
PER = 1568
S = 128
NEG = -3.0e38

_mesh = plsc.VectorSubcoreMesh(core_axis_name="c", subcore_axis_name="s")


def _insert3(cs, ci, t0s, t1s, t2s, t0i, t1i, t2i):
    gt0 = cs > t0s
    gt1 = cs > t1s
    gt2 = cs > t2s
    n0s = jnp.where(gt0, cs, t0s)
    n0i = jnp.where(gt0, ci, t0i)
    n1s = jnp.where(gt0, t0s, jnp.where(gt1, cs, t1s))
    n1i = jnp.where(gt0, t0i, jnp.where(gt1, ci, t1i))
    n2s = jnp.where(gt1, t1s, jnp.where(gt2, cs, t2s))
    n2i = jnp.where(gt1, t1i, jnp.where(gt2, ci, t2i))
    return n0s, n1s, n2s, n0i, n1i, n2i


@functools.partial(
    pl.kernel,
    out_type=(
        jax.ShapeDtypeStruct((NW, B + 1, D), jnp.float32),
        jax.ShapeDtypeStruct((NW, B + 1, 16), jnp.float32),
        jax.ShapeDtypeStruct((NW, B, 16), jnp.float32),
        jax.ShapeDtypeStruct((NW, B, 16), jnp.int32),
    ),
    mesh=_mesh,
    compiler_params=pltpu.CompilerParams(use_tc_tiling_on_sc=False),
    scratch_types=[
        pltpu.VMEM((S, D), jnp.float32),
        pltpu.VMEM((S,), jnp.int32),
        pltpu.VMEM((B + 1, D), jnp.float32),
        pltpu.VMEM((B + 1, 16), jnp.float32),
        pltpu.VMEM((B, 16), jnp.float32),
        pltpu.VMEM((B, 16), jnp.int32),
    ],
)
def _k1(x_hbm, bat_hbm,
        psum_hbm, pcnt_hbm, cands_hbm, candi_hbm,
        rows_v, idx_v, acc_v, cnt_v, ts_s, ts_i):
    cid = lax.axis_index("c")
    sid = lax.axis_index("s")
    wid = cid * 16 + sid
    start = wid * PER
    rows = jnp.minimum(PER, N - start)
    nc = (rows + (S - 1)) // S

    zeros16 = jnp.zeros((16,), jnp.float32)
    iota = lax.iota(jnp.int32, 16)

    def zrow(r, _):
        for c in range(D // 16):
            acc_v[r, pl.ds(c * 16, 16)] = zeros16
        cnt_v[r, :] = zeros16
        return 0

    lax.fori_loop(0, B + 1, zrow, 0)

    def init_body(r, _):
        ts_s[r, :] = jnp.full((16,), NEG, jnp.float32)
        ts_i[r, :] = jnp.zeros((16,), jnp.int32)
        return 0

    lax.fori_loop(0, B, init_body, 0)

    def chunk_body(j, _):
        cstart = jnp.minimum(start + j * S, start + rows - S)
        fresh_from = start + j * S
        pltpu.sync_copy(x_hbm.at[pl.ds(cstart, S)], rows_v)
        pltpu.sync_copy(bat_hbm.at[pl.ds(cstart, S)], idx_v)

        def grp_body(kk, _):
            goff = pl.multiple_of(kk * 16, 16)
            bv = idx_v[pl.ds(goff, 16)]
            gbase = cstart + kk * 16
            row0 = kk * 16
            for lane in range(16):
                b = bv[lane]
                g = gbase + lane
                fresh = g >= fresh_from
                beff = jnp.where(fresh, b, jnp.int32(B))
                onev = jnp.broadcast_to(
                    jnp.where(fresh, jnp.float32(1.0), jnp.float32(0.0)),
                    (16,))
                plsc.addupdate(cnt_v.at[beff], onev)
                row = row0 + lane
                sc = None
                for c in range(D // 16):
                    v = rows_v[row, pl.ds(c * 16, 16)]
                    plsc.addupdate(acc_v.at[beff, pl.ds(c * 16, 16)], v)
                    if c == D // 16 - 1:
                        sc = v[15]
                se = jnp.where(fresh, sc, jnp.float32(NEG))
                sv = ts_s[b, :]
                iv = ts_i[b, :]
                n0s, n1s, n2s, n0i, n1i, n2i = _insert3(
                    se, g, sv[0], sv[1], sv[2], iv[0], iv[1], iv[2])
                ns = jnp.where(iota == 0, n0s,
                               jnp.where(iota == 1, n1s,
                                         jnp.where(iota == 2, n2s, sv)))
                ni = jnp.where(iota == 0, n0i,
                               jnp.where(iota == 1, n1i,
                                         jnp.where(iota == 2, n2i, iv)))
                ts_s[b, :] = ns
                ts_i[b, :] = ni
            return 0

        lax.fori_loop(0, S // 16, grp_body, 0)
        return 0

    lax.fori_loop(0, nc, chunk_body, 0)

    pltpu.sync_copy(acc_v, psum_hbm.at[wid])
    pltpu.sync_copy(cnt_v, pcnt_hbm.at[wid])
    pltpu.sync_copy(ts_s, cands_hbm.at[wid])
    pltpu.sync_copy(ts_i, candi_hbm.at[wid])


@functools.partial(
    pl.kernel,
    out_type=jax.ShapeDtypeStruct((B, 5 * D), jnp.float32),
    mesh=_mesh,
    compiler_params=pltpu.CompilerParams(use_tc_tiling_on_sc=False),
    scratch_types=[
        pltpu.VMEM((NW, B, 16), jnp.float32),
        pltpu.VMEM((NW, B, 16), jnp.int32),
        pltpu.VMEM((NW, 1, D), jnp.float32),
        pltpu.VMEM((NW, 1, 16), jnp.float32),
        pltpu.VMEM((16,), jnp.int32),
        pltpu.VMEM((16, D), jnp.float32),
        pltpu.VMEM((5 * D,), jnp.float32),
        pltpu.SemaphoreType.DMA,
    ],
)
def _k2(x_hbm, psum_hbm, pcnt_hbm, cands_hbm, candi_hbm, out_hbm,
        cs_v, ci_v, psv, pcv, gi_v, grows_v, orow_v, sem):
    cid = lax.axis_index("c")
    sid = lax.axis_index("s")
    wid = cid * 16 + sid
    iota = lax.iota(jnp.int32, 16)
    zeros16 = jnp.zeros((16,), jnp.float32)
    pltpu.sync_copy(cands_hbm, cs_v)
    pltpu.sync_copy(candi_hbm, ci_v)

    def do_seg(seg):
        pltpu.sync_copy(psum_hbm.at[:, pl.ds(seg, 1), :], psv)
        pltpu.sync_copy(pcnt_hbm.at[:, pl.ds(seg, 1), :], pcv)

        def m_body(t, carry):
            t0s, t1s, t2s, t0i, t1i, t2i = carry
            csv = cs_v[t, seg, :]
            civ = ci_v[t, seg, :]
            for k in range(3):
                t0s, t1s, t2s, t0i, t1i, t2i = _insert3(
                    csv[k], civ[k], t0s, t1s, t2s, t0i, t1i, t2i)
            return (t0s, t1s, t2s, t0i, t1i, t2i)

        z = jnp.int32(0)
        ng = jnp.float32(NEG)
        t0s, t1s, t2s, t0i, t1i, t2i = lax.fori_loop(
            0, NW, m_body, (ng, ng, ng, z, z, z))

        def cnt_body(t, a):
            return a + pcv[t, 0, :]

        cnt = lax.fori_loop(0, NW, cnt_body, zeros16)
        cntc = jnp.maximum(cnt, jnp.float32(1.0))
        one = jnp.full((16,), 1.0, jnp.float32)
        v0 = jnp.where(cnt > 0.5, one, zeros16)
        v1 = jnp.where(cnt > 1.5, one, zeros16)
        v2 = jnp.where(cnt > 2.5, one, zeros16)

        gi_v[...] = jnp.where(iota == 0, t0i,
                              jnp.where(iota == 1, t1i,
                                        jnp.where(iota == 2, t2i, z)))
        pltpu.async_copy(x_hbm.at[gi_v], grows_v, sem).wait()

        def col_body(ccol, _):
            sl = pl.ds(pl.multiple_of(ccol * 16, 16), 16)

            def s_body(t, a):
                return a + psv[t, 0, sl]

            sv = lax.fori_loop(0, NW, s_body, zeros16)
            base = pl.multiple_of(ccol * 16, 16)
            orow_v[pl.ds(base, 16)] = sv / cntc
            orow_v[pl.ds(D + base, 16)] = sv
            orow_v[pl.ds(2 * D + base, 16)] = grows_v[0, sl] * v0
            orow_v[pl.ds(3 * D + base, 16)] = grows_v[1, sl] * v1
            orow_v[pl.ds(4 * D + base, 16)] = grows_v[2, sl] * v2
            return 0

        lax.fori_loop(0, D // 16, col_body, 0)
        pltpu.sync_copy(orow_v, out_hbm.at[seg])

    do_seg(wid * 2)
    do_seg(wid * 2 + 1)


def kernel(x, batch):
    bat = batch.astype(jnp.int32)
    psum, pcnt, cs, ci = _k1(x, bat)
    return _k2(x, psum, pcnt, cs, ci)


# speedup vs baseline: 7.0738x; 2.1055x over previous
import functools

import jax
import jax.numpy as jnp
from jax import lax
from jax.experimental import pallas as pl
from jax.experimental.pallas import tpu as pltpu
from jax.experimental.pallas import tpu_sc as plsc

N = 50000
D = 512
B = 64
NW = 32
PER = 1568
S = 64
NEG = -3.0e38

_mesh = plsc.VectorSubcoreMesh(core_axis_name="c", subcore_axis_name="s")


def _insert3(cs, ci, t0s, t1s, t2s, t0i, t1i, t2i):
    gt0 = cs > t0s
    gt1 = cs > t1s
    gt2 = cs > t2s
    n0s = jnp.where(gt0, cs, t0s)
    n0i = jnp.where(gt0, ci, t0i)
    n1s = jnp.where(gt0, t0s, jnp.where(gt1, cs, t1s))
    n1i = jnp.where(gt0, t0i, jnp.where(gt1, ci, t1i))
    n2s = jnp.where(gt1, t1s, jnp.where(gt2, cs, t2s))
    n2i = jnp.where(gt1, t1i, jnp.where(gt2, ci, t2i))
    return n0s, n1s, n2s, n0i, n1i, n2i


@functools.partial(
    pl.kernel,
    out_type=(
        jax.ShapeDtypeStruct((NW, B + 1, D), jnp.float32),
        jax.ShapeDtypeStruct((NW, B + 1, 16), jnp.float32),
        jax.ShapeDtypeStruct((NW, B, 16), jnp.float32),
        jax.ShapeDtypeStruct((NW, B, 16), jnp.int32),
    ),
    mesh=_mesh,
    compiler_params=pltpu.CompilerParams(use_tc_tiling_on_sc=False, needs_layout_passes=False),
    scratch_types=[
        pltpu.VMEM((S, D), jnp.float32),
        pltpu.VMEM((S, D), jnp.float32),
        pltpu.VMEM((S,), jnp.int32),
        pltpu.VMEM((S,), jnp.int32),
        pltpu.VMEM((B + 1, D), jnp.float32),
        pltpu.VMEM((B + 1, 16), jnp.float32),
        pltpu.VMEM((B, 16), jnp.float32),
        pltpu.VMEM((B, 16), jnp.int32),
        pltpu.SemaphoreType.DMA,
        pltpu.SemaphoreType.DMA,
        pltpu.SemaphoreType.DMA,
        pltpu.SemaphoreType.DMA,
    ],
)
def _k1(x_hbm, bat_hbm,
        psum_hbm, pcnt_hbm, cands_hbm, candi_hbm,
        rows0_v, rows1_v, idx0_v, idx1_v, acc_v, cnt_v, ts_s, ts_i,
        sem_r0, sem_r1, sem_i0, sem_i1):
    cid = lax.axis_index("c")
    sid = lax.axis_index("s")
    wid = cid * 16 + sid
    start = wid * PER
    rows = jnp.minimum(PER, N - start)
    nc = (rows + (S - 1)) // S

    zeros16 = jnp.zeros((16,), jnp.float32)
    iota = lax.iota(jnp.int32, 16)

    def zrow(r, _):
        for c in range(D // 16):
            acc_v[r, pl.ds(c * 16, 16)] = zeros16
        cnt_v[r, :] = zeros16
        return 0

    lax.fori_loop(0, B + 1, zrow, 0)

    def init_body(r, _):
        ts_s[r, :] = jnp.full((16,), NEG, jnp.float32)
        ts_i[r, :] = jnp.zeros((16,), jnp.int32)
        return 0

    lax.fori_loop(0, B, init_body, 0)

    def cs_of(j):
        return jnp.minimum(start + j * S, start + rows - S)

    def process(rows_v, idx_v, j):
        cstart = cs_of(j)
        fresh_from = start + j * S

        def grp(kk, _):
            row0 = kk * 16
            goff = pl.multiple_of(row0, 16)
            bv = idx_v[pl.ds(goff, 16)]
            gbase = cstart + row0
            rowids = jnp.broadcast_to(row0, (16,)) + iota
            svec = plsc.load_gather(
                rows_v, [rowids, jnp.full((16,), D - 1, jnp.int32)])
            gvec = jnp.broadcast_to(gbase, (16,)) + iota
            fresh_vec = gvec >= fresh_from
            s_eff = jnp.where(fresh_vec, svec, jnp.float32(NEG))
            thr = plsc.load_gather(
                ts_s, [bv, jnp.full((16,), 2, jnp.int32)])
            npass = plsc.all_reduce_population_count(s_eff > thr)[0]
            b0 = bv[0]
            uniform = jnp.logical_and(b0 == bv[15], gbase >= fresh_from)

            @pl.when(uniform)
            def _():
                def ucol(c4, _):
                    for u in range(4):
                        cbase = pl.multiple_of(c4 * 64 + u * 16, 16)
                        vs = [rows_v[row0 + l, pl.ds(cbase, 16)]
                              for l in range(16)]
                        while len(vs) > 1:
                            vs = [vs[i] + vs[i + 1]
                                  for i in range(0, len(vs), 2)]
                        plsc.addupdate(acc_v.at[b0, pl.ds(cbase, 16)], vs[0])
                    return 0

                lax.fori_loop(0, D // 64, ucol, 0)
                plsc.addupdate(cnt_v.at[b0],
                               jnp.full((16,), 16.0, jnp.float32))

            @pl.when(jnp.logical_not(uniform))
            def _():
                def lane_body(l, _):
                    lv = jnp.broadcast_to(l, (16,))
                    b = bv.at[lv].get(mode="promise_in_bounds")[0]
                    g = gbase + l
                    fresh = g >= fresh_from
                    beff = jnp.where(fresh, b, jnp.int32(B))
                    onev = jnp.broadcast_to(
                        jnp.where(fresh, jnp.float32(1.0), jnp.float32(0.0)),
                        (16,))
                    plsc.addupdate(cnt_v.at[beff], onev)
                    row = row0 + l

                    def fcol(c4, _):
                        for u in range(4):
                            cbase = pl.multiple_of(c4 * 64 + u * 16, 16)
                            v = rows_v[row, pl.ds(cbase, 16)]
                            plsc.addupdate(
                                acc_v.at[beff, pl.ds(cbase, 16)], v)
                        return 0

                    lax.fori_loop(0, D // 64, fcol, 0)
                    return 0

                lax.fori_loop(0, 16, lane_body, 0)

            @pl.when(npass > 0)
            def _():
                for l in range(16):
                    se = s_eff[l]
                    b = bv[l]
                    g = gbase + l
                    sv = ts_s[b, :]
                    iv = ts_i[b, :]
                    n0s, n1s, n2s, n0i, n1i, n2i = _insert3(
                        se, g, sv[0], sv[1], sv[2], iv[0], iv[1], iv[2])
                    ns = jnp.where(iota == 0, n0s,
                                   jnp.where(iota == 1, n1s,
                                             jnp.where(iota == 2, n2s, sv)))
                    ni = jnp.where(iota == 0, n0i,
                                   jnp.where(iota == 1, n1i,
                                             jnp.where(iota == 2, n2i, iv)))
                    ts_s[b, :] = ns
                    ts_i[b, :] = ni

            return 0

        lax.fori_loop(0, S // 16, grp, 0)

    pltpu.async_copy(x_hbm.at[pl.ds(cs_of(0), S)], rows0_v, sem_r0)
    pltpu.async_copy(bat_hbm.at[pl.ds(cs_of(0), S)], idx0_v, sem_i0)
    npairs = (nc + 1) // 2

    def pair_body(p, _):
        j0 = 2 * p
        pltpu.make_async_copy(
            x_hbm.at[pl.ds(cs_of(j0), S)], rows0_v, sem_r0).wait()
        pltpu.make_async_copy(
            bat_hbm.at[pl.ds(cs_of(j0), S)], idx0_v, sem_i0).wait()

        @pl.when(j0 + 1 < nc)
        def _():
            pltpu.async_copy(
                x_hbm.at[pl.ds(cs_of(j0 + 1), S)], rows1_v, sem_r1)
            pltpu.async_copy(
                bat_hbm.at[pl.ds(cs_of(j0 + 1), S)], idx1_v, sem_i1)

        process(rows0_v, idx0_v, j0)

        @pl.when(j0 + 1 < nc)
        def _():
            pltpu.make_async_copy(
                x_hbm.at[pl.ds(cs_of(j0 + 1), S)], rows1_v, sem_r1).wait()
            pltpu.make_async_copy(
                bat_hbm.at[pl.ds(cs_of(j0 + 1), S)], idx1_v, sem_i1).wait()

            @pl.when(j0 + 2 < nc)
            def _():
                pltpu.async_copy(
                    x_hbm.at[pl.ds(cs_of(j0 + 2), S)], rows0_v, sem_r0)
                pltpu.async_copy(
                    bat_hbm.at[pl.ds(cs_of(j0 + 2), S)], idx0_v, sem_i0)

            process(rows1_v, idx1_v, j0 + 1)

        return 0

    lax.fori_loop(0, npairs, pair_body, 0)

    pltpu.sync_copy(acc_v, psum_hbm.at[wid])
    pltpu.sync_copy(cnt_v, pcnt_hbm.at[wid])
    pltpu.sync_copy(ts_s, cands_hbm.at[wid])
    pltpu.sync_copy(ts_i, candi_hbm.at[wid])


@functools.partial(
    pl.kernel,
    out_type=jax.ShapeDtypeStruct((B, 5 * D), jnp.float32),
    mesh=_mesh,
    compiler_params=pltpu.CompilerParams(use_tc_tiling_on_sc=False, needs_layout_passes=False),
    scratch_types=[
        pltpu.VMEM((NW, B, 16), jnp.float32),
        pltpu.VMEM((NW, B, 16), jnp.int32),
        pltpu.VMEM((NW, 1, D), jnp.float32),
        pltpu.VMEM((NW, 1, 16), jnp.float32),
        pltpu.VMEM((16,), jnp.int32),
        pltpu.VMEM((16, D), jnp.float32),
        pltpu.VMEM((5 * D,), jnp.float32),
        pltpu.SemaphoreType.DMA,
    ],
)
def _k2(x_hbm, psum_hbm, pcnt_hbm, cands_hbm, candi_hbm, out_hbm,
        cs_v, ci_v, psv, pcv, gi_v, grows_v, orow_v, sem):
    cid = lax.axis_index("c")
    sid = lax.axis_index("s")
    wid = cid * 16 + sid
    iota = lax.iota(jnp.int32, 16)
    zeros16 = jnp.zeros((16,), jnp.float32)
    pltpu.sync_copy(cands_hbm, cs_v)
    pltpu.sync_copy(candi_hbm, ci_v)

    def do_seg(seg):
        pltpu.sync_copy(psum_hbm.at[:, pl.ds(seg, 1), :], psv)
        pltpu.sync_copy(pcnt_hbm.at[:, pl.ds(seg, 1), :], pcv)

        def m_body(t, carry):
            t0s, t1s, t2s, t0i, t1i, t2i = carry
            csv = cs_v[t, seg, :]
            civ = ci_v[t, seg, :]
            for k in range(3):
                t0s, t1s, t2s, t0i, t1i, t2i = _insert3(
                    csv[k], civ[k], t0s, t1s, t2s, t0i, t1i, t2i)
            return (t0s, t1s, t2s, t0i, t1i, t2i)

        z = jnp.int32(0)
        ng = jnp.float32(NEG)
        t0s, t1s, t2s, t0i, t1i, t2i = lax.fori_loop(
            0, NW, m_body, (ng, ng, ng, z, z, z))

        def cnt_body(t, a):
            return a + pcv[t, 0, :]

        cnt = lax.fori_loop(0, NW, cnt_body, zeros16)
        cntc = jnp.maximum(cnt, jnp.float32(1.0))
        one = jnp.full((16,), 1.0, jnp.float32)
        v0 = jnp.where(cnt > 0.5, one, zeros16)
        v1 = jnp.where(cnt > 1.5, one, zeros16)
        v2 = jnp.where(cnt > 2.5, one, zeros16)

        gi_v[...] = jnp.where(iota == 0, t0i,
                              jnp.where(iota == 1, t1i,
                                        jnp.where(iota == 2, t2i, z)))
        pltpu.async_copy(x_hbm.at[gi_v], grows_v, sem).wait()

        def col_body(ccol, _):
            sl = pl.ds(pl.multiple_of(ccol * 16, 16), 16)

            def s_body(t, a):
                return a + psv[t, 0, sl]

            sv = lax.fori_loop(0, NW, s_body, zeros16)
            base = pl.multiple_of(ccol * 16, 16)
            orow_v[pl.ds(base, 16)] = sv / cntc
            orow_v[pl.ds(D + base, 16)] = sv
            orow_v[pl.ds(2 * D + base, 16)] = grows_v[0, sl] * v0
            orow_v[pl.ds(3 * D + base, 16)] = grows_v[1, sl] * v1
            orow_v[pl.ds(4 * D + base, 16)] = grows_v[2, sl] * v2
            return 0

        lax.fori_loop(0, D // 16, col_body, 0)
        pltpu.sync_copy(orow_v, out_hbm.at[seg])

    do_seg(wid * 2)
    do_seg(wid * 2 + 1)


def kernel(x, batch):
    bat = batch.astype(jnp.int32)
    psum, pcnt, cs, ci = _k1(x, bat)
    return _k2(x, psum, pcnt, cs, ci)


# speedup vs baseline: 7.2278x; 1.0218x over previous
import functools

import jax
import jax.numpy as jnp
from jax import lax
from jax.experimental import pallas as pl
from jax.experimental.pallas import tpu as pltpu
from jax.experimental.pallas import tpu_sc as plsc

N = 50000
D = 512
B = 64
NW = 32
PER = 1568
S = 64
NEG = -3.0e38

_mesh = plsc.VectorSubcoreMesh(core_axis_name="c", subcore_axis_name="s")


def _insert3(cs, ci, t0s, t1s, t2s, t0i, t1i, t2i):
    gt0 = cs > t0s
    gt1 = cs > t1s
    gt2 = cs > t2s
    n0s = jnp.where(gt0, cs, t0s)
    n0i = jnp.where(gt0, ci, t0i)
    n1s = jnp.where(gt0, t0s, jnp.where(gt1, cs, t1s))
    n1i = jnp.where(gt0, t0i, jnp.where(gt1, ci, t1i))
    n2s = jnp.where(gt1, t1s, jnp.where(gt2, cs, t2s))
    n2i = jnp.where(gt1, t1i, jnp.where(gt2, ci, t2i))
    return n0s, n1s, n2s, n0i, n1i, n2i


@functools.partial(
    pl.kernel,
    out_type=(
        jax.ShapeDtypeStruct((NW, B + 1, D), jnp.float32),
        jax.ShapeDtypeStruct((NW, B + 1, 16), jnp.float32),
        jax.ShapeDtypeStruct((NW, B, 16), jnp.float32),
        jax.ShapeDtypeStruct((NW, B, 16), jnp.int32),
    ),
    mesh=_mesh,
    compiler_params=pltpu.CompilerParams(use_tc_tiling_on_sc=False, needs_layout_passes=False),
    scratch_types=[
        pltpu.VMEM((S, D), jnp.float32),
        pltpu.VMEM((S, D), jnp.float32),
        pltpu.VMEM((S,), jnp.int32),
        pltpu.VMEM((S,), jnp.int32),
        pltpu.VMEM((B + 1, D), jnp.float32),
        pltpu.VMEM((B + 1, 16), jnp.float32),
        pltpu.VMEM((B, 16), jnp.float32),
        pltpu.VMEM((B, 16), jnp.int32),
        pltpu.SemaphoreType.DMA,
        pltpu.SemaphoreType.DMA,
        pltpu.SemaphoreType.DMA,
        pltpu.SemaphoreType.DMA,
    ],
)
def _k1(x_hbm, bat_hbm,
        psum_hbm, pcnt_hbm, cands_hbm, candi_hbm,
        rows0_v, rows1_v, idx0_v, idx1_v, acc_v, cnt_v, ts_s, ts_i,
        sem_r0, sem_r1, sem_i0, sem_i1):
    cid = lax.axis_index("c")
    sid = lax.axis_index("s")
    wid = cid * 16 + sid
    start = wid * PER
    rows = jnp.minimum(PER, N - start)
    nc = (rows + (S - 1)) // S

    zeros16 = jnp.zeros((16,), jnp.float32)
    iota = lax.iota(jnp.int32, 16)

    def zrow(r, _):
        for c in range(D // 16):
            acc_v[r, pl.ds(c * 16, 16)] = zeros16
        cnt_v[r, :] = zeros16
        return 0

    lax.fori_loop(0, B + 1, zrow, 0)

    def init_body(r, _):
        ts_s[r, :] = jnp.full((16,), NEG, jnp.float32)
        ts_i[r, :] = jnp.zeros((16,), jnp.int32)
        return 0

    lax.fori_loop(0, B, init_body, 0)

    def cs_of(j):
        return jnp.minimum(start + j * S, start + rows - S)

    def process(rows_v, idx_v, j):
        cstart = cs_of(j)
        fresh_from = start + j * S

        def grp(kk, _):
            row0 = kk * 16
            goff = pl.multiple_of(row0, 16)
            bv = idx_v[pl.ds(goff, 16)]
            gbase = cstart + row0
            rowids = jnp.broadcast_to(row0, (16,)) + iota
            svec = plsc.load_gather(
                rows_v, [rowids, jnp.full((16,), D - 1, jnp.int32)])
            gvec = jnp.broadcast_to(gbase, (16,)) + iota
            fresh_vec = gvec >= fresh_from
            s_eff = jnp.where(fresh_vec, svec, jnp.float32(NEG))
            thr = plsc.load_gather(
                ts_s, [bv, jnp.full((16,), 2, jnp.int32)])
            npass = plsc.all_reduce_population_count(s_eff > thr)[0]
            b0 = bv[0]
            uniform = jnp.logical_and(b0 == bv[15], gbase >= fresh_from)

            @pl.when(uniform)
            def _():
                def ucol(c4, _):
                    for u in range(4):
                        cbase = pl.multiple_of(c4 * 64 + u * 16, 16)
                        vs = [rows_v[row0 + l, pl.ds(cbase, 16)]
                              for l in range(16)]
                        while len(vs) > 1:
                            vs = [vs[i] + vs[i + 1]
                                  for i in range(0, len(vs), 2)]
                        plsc.addupdate(acc_v.at[b0, pl.ds(cbase, 16)], vs[0])
                    return 0

                lax.fori_loop(0, D // 64, ucol, 0)
                plsc.addupdate(cnt_v.at[b0],
                               jnp.full((16,), 16.0, jnp.float32))

            @pl.when(jnp.logical_not(uniform))
            def _():
                def lane_body(l, _):
                    lv = jnp.broadcast_to(l, (16,))
                    b = bv.at[lv].get(mode="promise_in_bounds")[0]
                    g = gbase + l
                    fresh = g >= fresh_from
                    beff = jnp.where(fresh, b, jnp.int32(B))
                    onev = jnp.broadcast_to(
                        jnp.where(fresh, jnp.float32(1.0), jnp.float32(0.0)),
                        (16,))
                    plsc.addupdate(cnt_v.at[beff], onev)
                    row = row0 + l

                    def fcol(c4, _):
                        for u in range(4):
                            cbase = pl.multiple_of(c4 * 64 + u * 16, 16)
                            v = rows_v[row, pl.ds(cbase, 16)]
                            plsc.addupdate(
                                acc_v.at[beff, pl.ds(cbase, 16)], v)
                        return 0

                    lax.fori_loop(0, D // 64, fcol, 0)
                    return 0

                lax.fori_loop(0, 16, lane_body, 0)

            @pl.when(npass > 0)
            def _():
                for l in range(16):
                    se = s_eff[l]
                    b = bv[l]
                    g = gbase + l
                    sv = ts_s[b, :]
                    iv = ts_i[b, :]
                    n0s, n1s, n2s, n0i, n1i, n2i = _insert3(
                        se, g, sv[0], sv[1], sv[2], iv[0], iv[1], iv[2])
                    ns = jnp.where(iota == 0, n0s,
                                   jnp.where(iota == 1, n1s,
                                             jnp.where(iota == 2, n2s, sv)))
                    ni = jnp.where(iota == 0, n0i,
                                   jnp.where(iota == 1, n1i,
                                             jnp.where(iota == 2, n2i, iv)))
                    ts_s[b, :] = ns
                    ts_i[b, :] = ni

            return 0

        lax.fori_loop(0, S // 16, grp, 0)

    pltpu.async_copy(x_hbm.at[pl.ds(cs_of(0), S)], rows0_v, sem_r0)
    pltpu.async_copy(bat_hbm.at[pl.ds(cs_of(0), S)], idx0_v, sem_i0)
    npairs = (nc + 1) // 2

    def pair_body(p, _):
        j0 = 2 * p
        pltpu.make_async_copy(
            x_hbm.at[pl.ds(cs_of(j0), S)], rows0_v, sem_r0).wait()
        pltpu.make_async_copy(
            bat_hbm.at[pl.ds(cs_of(j0), S)], idx0_v, sem_i0).wait()

        @pl.when(j0 + 1 < nc)
        def _():
            pltpu.async_copy(
                x_hbm.at[pl.ds(cs_of(j0 + 1), S)], rows1_v, sem_r1)
            pltpu.async_copy(
                bat_hbm.at[pl.ds(cs_of(j0 + 1), S)], idx1_v, sem_i1)

        process(rows0_v, idx0_v, j0)

        @pl.when(j0 + 1 < nc)
        def _():
            pltpu.make_async_copy(
                x_hbm.at[pl.ds(cs_of(j0 + 1), S)], rows1_v, sem_r1).wait()
            pltpu.make_async_copy(
                bat_hbm.at[pl.ds(cs_of(j0 + 1), S)], idx1_v, sem_i1).wait()

            @pl.when(j0 + 2 < nc)
            def _():
                pltpu.async_copy(
                    x_hbm.at[pl.ds(cs_of(j0 + 2), S)], rows0_v, sem_r0)
                pltpu.async_copy(
                    bat_hbm.at[pl.ds(cs_of(j0 + 2), S)], idx0_v, sem_i0)

            process(rows1_v, idx1_v, j0 + 1)

        return 0

    lax.fori_loop(0, npairs, pair_body, 0)

    pltpu.sync_copy(acc_v, psum_hbm.at[wid])
    pltpu.sync_copy(cnt_v, pcnt_hbm.at[wid])
    pltpu.sync_copy(ts_s, cands_hbm.at[wid])
    pltpu.sync_copy(ts_i, candi_hbm.at[wid])


@functools.partial(
    pl.kernel,
    out_type=jax.ShapeDtypeStruct((B, 5 * D), jnp.float32),
    mesh=_mesh,
    compiler_params=pltpu.CompilerParams(use_tc_tiling_on_sc=False, needs_layout_passes=False),
    scratch_types=[
        pltpu.VMEM((NW, 1, 16), jnp.float32),
        pltpu.VMEM((NW, 1, 16), jnp.int32),
        pltpu.VMEM((NW, 1, D), jnp.float32),
        pltpu.VMEM((NW, 1, 16), jnp.float32),
        pltpu.VMEM((16,), jnp.int32),
        pltpu.VMEM((16, D), jnp.float32),
        pltpu.VMEM((5 * D,), jnp.float32),
        pltpu.SemaphoreType.DMA,
    ],
)
def _k2(x_hbm, psum_hbm, pcnt_hbm, cands_hbm, candi_hbm, out_hbm,
        cs_v, ci_v, psv, pcv, gi_v, grows_v, orow_v, sem):
    cid = lax.axis_index("c")
    sid = lax.axis_index("s")
    wid = cid * 16 + sid
    iota = lax.iota(jnp.int32, 16)
    zeros16 = jnp.zeros((16,), jnp.float32)

    def do_seg(seg):
        pltpu.sync_copy(cands_hbm.at[:, pl.ds(seg, 1), :], cs_v)
        pltpu.sync_copy(candi_hbm.at[:, pl.ds(seg, 1), :], ci_v)
        pltpu.sync_copy(psum_hbm.at[:, pl.ds(seg, 1), :], psv)
        pltpu.sync_copy(pcnt_hbm.at[:, pl.ds(seg, 1), :], pcv)

        def m_body(t, carry):
            t0s, t1s, t2s, t0i, t1i, t2i = carry
            csv = cs_v[t, 0, :]
            civ = ci_v[t, 0, :]
            for k in range(3):
                t0s, t1s, t2s, t0i, t1i, t2i = _insert3(
                    csv[k], civ[k], t0s, t1s, t2s, t0i, t1i, t2i)
            return (t0s, t1s, t2s, t0i, t1i, t2i)

        z = jnp.int32(0)
        ng = jnp.float32(NEG)
        t0s, t1s, t2s, t0i, t1i, t2i = lax.fori_loop(
            0, NW, m_body, (ng, ng, ng, z, z, z))

        def cnt_body(t, a):
            return a + pcv[t, 0, :]

        cnt = lax.fori_loop(0, NW, cnt_body, zeros16)
        cntc = jnp.maximum(cnt, jnp.float32(1.0))
        one = jnp.full((16,), 1.0, jnp.float32)
        v0 = jnp.where(cnt > 0.5, one, zeros16)
        v1 = jnp.where(cnt > 1.5, one, zeros16)
        v2 = jnp.where(cnt > 2.5, one, zeros16)

        gi_v[...] = jnp.where(iota == 0, t0i,
                              jnp.where(iota == 1, t1i,
                                        jnp.where(iota == 2, t2i, z)))
        pltpu.async_copy(x_hbm.at[gi_v], grows_v, sem).wait()

        def col_body(c4, _):
            bases = [pl.multiple_of(c4 * 64 + u * 16, 16) for u in range(4)]

            def s_body(t, accs):
                return tuple(a + psv[t, 0, pl.ds(bases[u], 16)]
                             for u, a in enumerate(accs))

            svs = lax.fori_loop(0, NW, s_body, (zeros16,) * 4)
            for u in range(4):
                base = bases[u]
                sl = pl.ds(base, 16)
                sv = svs[u]
                orow_v[pl.ds(base, 16)] = sv / cntc
                orow_v[pl.ds(D + base, 16)] = sv
                orow_v[pl.ds(2 * D + base, 16)] = grows_v[0, sl] * v0
                orow_v[pl.ds(3 * D + base, 16)] = grows_v[1, sl] * v1
                orow_v[pl.ds(4 * D + base, 16)] = grows_v[2, sl] * v2
            return 0

        lax.fori_loop(0, D // 64, col_body, 0)
        pltpu.sync_copy(orow_v, out_hbm.at[seg])

    do_seg(wid * 2)
    do_seg(wid * 2 + 1)


def kernel(x, batch):
    bat = batch.astype(jnp.int32)
    psum, pcnt, cs, ci = _k1(x, bat)
    return _k2(x, psum, pcnt, cs, ci)


# speedup vs baseline: 10.7246x; 1.4838x over previous
import functools

import jax
import jax.numpy as jnp
from jax import lax
from jax.experimental import pallas as pl
from jax.experimental.pallas import tpu as pltpu
from jax.experimental.pallas import tpu_sc as plsc

N = 50000
D = 512
B = 64
NW = 32
PER = 1568
S = 64
NEG = -3.0e38

_mesh = plsc.VectorSubcoreMesh(core_axis_name="c", subcore_axis_name="s")


def _insert3(cs, ci, t0s, t1s, t2s, t0i, t1i, t2i):
    gt0 = cs > t0s
    gt1 = cs > t1s
    gt2 = cs > t2s
    n0s = jnp.where(gt0, cs, t0s)
    n0i = jnp.where(gt0, ci, t0i)
    n1s = jnp.where(gt0, t0s, jnp.where(gt1, cs, t1s))
    n1i = jnp.where(gt0, t0i, jnp.where(gt1, ci, t1i))
    n2s = jnp.where(gt1, t1s, jnp.where(gt2, cs, t2s))
    n2i = jnp.where(gt1, t1i, jnp.where(gt2, ci, t2i))
    return n0s, n1s, n2s, n0i, n1i, n2i


@functools.partial(
    pl.kernel,
    out_type=(
        jax.ShapeDtypeStruct((NW, B + 1, D), jnp.float32),
        jax.ShapeDtypeStruct((NW, B + 1, 16), jnp.float32),
        jax.ShapeDtypeStruct((NW, B, 16), jnp.float32),
        jax.ShapeDtypeStruct((NW, B, 16), jnp.int32),
    ),
    mesh=_mesh,
    compiler_params=pltpu.CompilerParams(use_tc_tiling_on_sc=True, needs_layout_passes=False),
    scratch_types=[
        pltpu.VMEM((S, D), jnp.float32),
        pltpu.VMEM((S, D), jnp.float32),
        pltpu.VMEM((S,), jnp.int32),
        pltpu.VMEM((S,), jnp.int32),
        pltpu.VMEM((B + 1, D), jnp.float32),
        pltpu.VMEM((B + 1, 16), jnp.float32),
        pltpu.VMEM((B, 16), jnp.float32),
        pltpu.VMEM((B, 16), jnp.int32),
        pltpu.SemaphoreType.DMA,
        pltpu.SemaphoreType.DMA,
        pltpu.SemaphoreType.DMA,
        pltpu.SemaphoreType.DMA,
    ],
)
def _k1(x_hbm, bat_hbm,
        psum_hbm, pcnt_hbm, cands_hbm, candi_hbm,
        rows0_v, rows1_v, idx0_v, idx1_v, acc_v, cnt_v, ts_s, ts_i,
        sem_r0, sem_r1, sem_i0, sem_i1):
    cid = lax.axis_index("c")
    sid = lax.axis_index("s")
    wid = cid * 16 + sid
    start = wid * PER
    rows = jnp.minimum(PER, N - start)
    nc = (rows + (S - 1)) // S

    zeros16 = jnp.zeros((16,), jnp.float32)
    iota = lax.iota(jnp.int32, 16)

    def zrow(r, _):
        for c in range(D // 16):
            acc_v[r, pl.ds(c * 16, 16)] = zeros16
        cnt_v[r, :] = zeros16
        return 0

    lax.fori_loop(0, B + 1, zrow, 0)

    def init_body(r, _):
        ts_s[r, :] = jnp.full((16,), NEG, jnp.float32)
        ts_i[r, :] = jnp.zeros((16,), jnp.int32)
        return 0

    lax.fori_loop(0, B, init_body, 0)

    def cs_of(j):
        return jnp.minimum(start + j * S, start + rows - S)

    def process(rows_v, idx_v, j):
        cstart = cs_of(j)
        fresh_from = start + j * S

        def grp(kk, _):
            row0 = kk * 16
            goff = pl.multiple_of(row0, 16)
            bv = idx_v[pl.ds(goff, 16)]
            gbase = cstart + row0
            rowids = jnp.broadcast_to(row0, (16,)) + iota
            svec = plsc.load_gather(
                rows_v, [rowids, jnp.full((16,), D - 1, jnp.int32)])
            gvec = jnp.broadcast_to(gbase, (16,)) + iota
            fresh_vec = gvec >= fresh_from
            s_eff = jnp.where(fresh_vec, svec, jnp.float32(NEG))
            thr = plsc.load_gather(
                ts_s, [bv, jnp.full((16,), 2, jnp.int32)])
            npass = plsc.all_reduce_population_count(s_eff > thr)[0]
            b0 = bv[0]
            uniform = jnp.logical_and(b0 == bv[15], gbase >= fresh_from)

            @pl.when(uniform)
            def _():
                def ucol(c4, _):
                    for u in range(4):
                        cbase = pl.multiple_of(c4 * 64 + u * 16, 16)
                        vs = [rows_v[row0 + l, pl.ds(cbase, 16)]
                              for l in range(16)]
                        while len(vs) > 1:
                            vs = [vs[i] + vs[i + 1]
                                  for i in range(0, len(vs), 2)]
                        plsc.addupdate(acc_v.at[b0, pl.ds(cbase, 16)], vs[0])
                    return 0

                lax.fori_loop(0, D // 64, ucol, 0)
                plsc.addupdate(cnt_v.at[b0],
                               jnp.full((16,), 16.0, jnp.float32))

            @pl.when(jnp.logical_not(uniform))
            def _():
                def lane_body(l, _):
                    lv = jnp.broadcast_to(l, (16,))
                    b = bv.at[lv].get(mode="promise_in_bounds")[0]
                    g = gbase + l
                    fresh = g >= fresh_from
                    beff = jnp.where(fresh, b, jnp.int32(B))
                    onev = jnp.broadcast_to(
                        jnp.where(fresh, jnp.float32(1.0), jnp.float32(0.0)),
                        (16,))
                    plsc.addupdate(cnt_v.at[beff], onev)
                    row = row0 + l

                    def fcol(c4, _):
                        for u in range(4):
                            cbase = pl.multiple_of(c4 * 64 + u * 16, 16)
                            v = rows_v[row, pl.ds(cbase, 16)]
                            plsc.addupdate(
                                acc_v.at[beff, pl.ds(cbase, 16)], v)
                        return 0

                    lax.fori_loop(0, D // 64, fcol, 0)
                    return 0

                lax.fori_loop(0, 16, lane_body, 0)

            @pl.when(npass > 0)
            def _():
                for l in range(16):
                    se = s_eff[l]
                    b = bv[l]
                    g = gbase + l
                    sv = ts_s[b, :]
                    iv = ts_i[b, :]
                    n0s, n1s, n2s, n0i, n1i, n2i = _insert3(
                        se, g, sv[0], sv[1], sv[2], iv[0], iv[1], iv[2])
                    ns = jnp.where(iota == 0, n0s,
                                   jnp.where(iota == 1, n1s,
                                             jnp.where(iota == 2, n2s, sv)))
                    ni = jnp.where(iota == 0, n0i,
                                   jnp.where(iota == 1, n1i,
                                             jnp.where(iota == 2, n2i, iv)))
                    ts_s[b, :] = ns
                    ts_i[b, :] = ni

            return 0

        lax.fori_loop(0, S // 16, grp, 0)

    pltpu.async_copy(x_hbm.at[pl.ds(cs_of(0), S)], rows0_v, sem_r0)
    pltpu.async_copy(bat_hbm.at[pl.ds(cs_of(0), S)], idx0_v, sem_i0)
    npairs = (nc + 1) // 2

    def pair_body(p, _):
        j0 = 2 * p
        pltpu.make_async_copy(
            x_hbm.at[pl.ds(cs_of(j0), S)], rows0_v, sem_r0).wait()
        pltpu.make_async_copy(
            bat_hbm.at[pl.ds(cs_of(j0), S)], idx0_v, sem_i0).wait()

        @pl.when(j0 + 1 < nc)
        def _():
            pltpu.async_copy(
                x_hbm.at[pl.ds(cs_of(j0 + 1), S)], rows1_v, sem_r1)
            pltpu.async_copy(
                bat_hbm.at[pl.ds(cs_of(j0 + 1), S)], idx1_v, sem_i1)

        process(rows0_v, idx0_v, j0)

        @pl.when(j0 + 1 < nc)
        def _():
            pltpu.make_async_copy(
                x_hbm.at[pl.ds(cs_of(j0 + 1), S)], rows1_v, sem_r1).wait()
            pltpu.make_async_copy(
                bat_hbm.at[pl.ds(cs_of(j0 + 1), S)], idx1_v, sem_i1).wait()

            @pl.when(j0 + 2 < nc)
            def _():
                pltpu.async_copy(
                    x_hbm.at[pl.ds(cs_of(j0 + 2), S)], rows0_v, sem_r0)
                pltpu.async_copy(
                    bat_hbm.at[pl.ds(cs_of(j0 + 2), S)], idx0_v, sem_i0)

            process(rows1_v, idx1_v, j0 + 1)

        return 0

    lax.fori_loop(0, npairs, pair_body, 0)

    pltpu.sync_copy(acc_v, psum_hbm.at[wid])
    pltpu.sync_copy(cnt_v, pcnt_hbm.at[wid])
    pltpu.sync_copy(ts_s, cands_hbm.at[wid])
    pltpu.sync_copy(ts_i, candi_hbm.at[wid])


@functools.partial(
    pl.kernel,
    out_type=jax.ShapeDtypeStruct((B, 5 * D), jnp.float32),
    mesh=_mesh,
    compiler_params=pltpu.CompilerParams(use_tc_tiling_on_sc=True, needs_layout_passes=False),
    scratch_types=[
        pltpu.VMEM((NW, 1, 16), jnp.float32),
        pltpu.VMEM((NW, 1, 16), jnp.int32),
        pltpu.VMEM((NW, 1, D), jnp.float32),
        pltpu.VMEM((NW, 1, 16), jnp.float32),
        pltpu.VMEM((16,), jnp.int32),
        pltpu.VMEM((16, D), jnp.float32),
        pltpu.VMEM((5 * D,), jnp.float32),
        pltpu.SemaphoreType.DMA,
    ],
)
def _k2(x_hbm, psum_hbm, pcnt_hbm, cands_hbm, candi_hbm, out_hbm,
        cs_v, ci_v, psv, pcv, gi_v, grows_v, orow_v, sem):
    cid = lax.axis_index("c")
    sid = lax.axis_index("s")
    wid = cid * 16 + sid
    iota = lax.iota(jnp.int32, 16)
    zeros16 = jnp.zeros((16,), jnp.float32)

    def do_seg(seg):
        pltpu.sync_copy(cands_hbm.at[:, pl.ds(seg, 1), :], cs_v)
        pltpu.sync_copy(candi_hbm.at[:, pl.ds(seg, 1), :], ci_v)
        pltpu.sync_copy(psum_hbm.at[:, pl.ds(seg, 1), :], psv)
        pltpu.sync_copy(pcnt_hbm.at[:, pl.ds(seg, 1), :], pcv)

        def m_body(t, carry):
            t0s, t1s, t2s, t0i, t1i, t2i = carry
            csv = cs_v[t, 0, :]
            civ = ci_v[t, 0, :]
            for k in range(3):
                t0s, t1s, t2s, t0i, t1i, t2i = _insert3(
                    csv[k], civ[k], t0s, t1s, t2s, t0i, t1i, t2i)
            return (t0s, t1s, t2s, t0i, t1i, t2i)

        z = jnp.int32(0)
        ng = jnp.float32(NEG)
        t0s, t1s, t2s, t0i, t1i, t2i = lax.fori_loop(
            0, NW, m_body, (ng, ng, ng, z, z, z))

        def cnt_body(t, a):
            return a + pcv[t, 0, :]

        cnt = lax.fori_loop(0, NW, cnt_body, zeros16)
        cntc = jnp.maximum(cnt, jnp.float32(1.0))
        one = jnp.full((16,), 1.0, jnp.float32)
        v0 = jnp.where(cnt > 0.5, one, zeros16)
        v1 = jnp.where(cnt > 1.5, one, zeros16)
        v2 = jnp.where(cnt > 2.5, one, zeros16)

        gi_v[...] = jnp.where(iota == 0, t0i,
                              jnp.where(iota == 1, t1i,
                                        jnp.where(iota == 2, t2i, z)))
        pltpu.async_copy(x_hbm.at[gi_v], grows_v, sem).wait()

        def col_body(c4, _):
            bases = [pl.multiple_of(c4 * 64 + u * 16, 16) for u in range(4)]

            def s_body(t, accs):
                return tuple(a + psv[t, 0, pl.ds(bases[u], 16)]
                             for u, a in enumerate(accs))

            svs = lax.fori_loop(0, NW, s_body, (zeros16,) * 4)
            for u in range(4):
                base = bases[u]
                sl = pl.ds(base, 16)
                sv = svs[u]
                orow_v[pl.ds(base, 16)] = sv / cntc
                orow_v[pl.ds(D + base, 16)] = sv
                orow_v[pl.ds(2 * D + base, 16)] = grows_v[0, sl] * v0
                orow_v[pl.ds(3 * D + base, 16)] = grows_v[1, sl] * v1
                orow_v[pl.ds(4 * D + base, 16)] = grows_v[2, sl] * v2
            return 0

        lax.fori_loop(0, D // 64, col_body, 0)
        pltpu.sync_copy(orow_v, out_hbm.at[seg])

    do_seg(wid * 2)
    do_seg(wid * 2 + 1)


def kernel(x, batch):
    bat = batch.astype(jnp.int32)
    psum, pcnt, cs, ci = _k1(x, bat)
    return _k2(x, psum, pcnt, cs, ci)


# speedup vs baseline: 10.9738x; 1.0232x over previous
import functools

import jax
import jax.numpy as jnp
from jax import lax
from jax.experimental import pallas as pl
from jax.experimental.pallas import tpu as pltpu
from jax.experimental.pallas import tpu_sc as plsc

N = 50000
D = 512
B = 64
NW = 32
PER = 1568
S = 64
NEG = -3.0e38

_mesh = plsc.VectorSubcoreMesh(core_axis_name="c", subcore_axis_name="s")


def _insert3(cs, ci, t0s, t1s, t2s, t0i, t1i, t2i):
    gt0 = cs > t0s
    gt1 = cs > t1s
    gt2 = cs > t2s
    n0s = jnp.where(gt0, cs, t0s)
    n0i = jnp.where(gt0, ci, t0i)
    n1s = jnp.where(gt0, t0s, jnp.where(gt1, cs, t1s))
    n1i = jnp.where(gt0, t0i, jnp.where(gt1, ci, t1i))
    n2s = jnp.where(gt1, t1s, jnp.where(gt2, cs, t2s))
    n2i = jnp.where(gt1, t1i, jnp.where(gt2, ci, t2i))
    return n0s, n1s, n2s, n0i, n1i, n2i


@functools.partial(
    pl.kernel,
    out_type=(
        jax.ShapeDtypeStruct((NW, B + 1, D), jnp.float32),
        jax.ShapeDtypeStruct((NW, B + 1, 16), jnp.float32),
        jax.ShapeDtypeStruct((NW, B, 16), jnp.float32),
        jax.ShapeDtypeStruct((NW, B, 16), jnp.int32),
    ),
    mesh=_mesh,
    compiler_params=pltpu.CompilerParams(use_tc_tiling_on_sc=True, needs_layout_passes=False),
    scratch_types=[
        pltpu.VMEM((S, D), jnp.float32),
        pltpu.VMEM((S, D), jnp.float32),
        pltpu.VMEM((S,), jnp.int32),
        pltpu.VMEM((S,), jnp.int32),
        pltpu.VMEM((B + 1, D), jnp.float32),
        pltpu.VMEM((B + 1, 16), jnp.float32),
        pltpu.VMEM((B, 16), jnp.float32),
        pltpu.VMEM((B, 16), jnp.int32),
        pltpu.SemaphoreType.DMA,
        pltpu.SemaphoreType.DMA,
        pltpu.SemaphoreType.DMA,
        pltpu.SemaphoreType.DMA,
    ],
)
def _k1(x_hbm, bat_hbm,
        psum_hbm, pcnt_hbm, cands_hbm, candi_hbm,
        rows0_v, rows1_v, idx0_v, idx1_v, acc_v, cnt_v, ts_s, ts_i,
        sem_r0, sem_r1, sem_i0, sem_i1):
    cid = lax.axis_index("c")
    sid = lax.axis_index("s")
    wid = cid * 16 + sid
    start = wid * PER
    rows = jnp.minimum(PER, N - start)
    nc = (rows + (S - 1)) // S

    zeros16 = jnp.zeros((16,), jnp.float32)
    iota = lax.iota(jnp.int32, 16)

    def zrow(r, _):
        for c in range(D // 16):
            acc_v[r, pl.ds(c * 16, 16)] = zeros16
        cnt_v[r, :] = zeros16
        return 0

    lax.fori_loop(0, B + 1, zrow, 0)

    def init_body(r, _):
        ts_s[r, :] = jnp.full((16,), NEG, jnp.float32)
        ts_i[r, :] = jnp.zeros((16,), jnp.int32)
        return 0

    lax.fori_loop(0, B, init_body, 0)

    def cs_of(j):
        return jnp.minimum(start + j * S, start + rows - S)

    def process(rows_v, idx_v, j):
        cstart = cs_of(j)
        fresh_from = start + j * S

        def grp(kk, _):
            row0 = kk * 16
            goff = pl.multiple_of(row0, 16)
            bv = idx_v[pl.ds(goff, 16)]
            gbase = cstart + row0
            rowids = jnp.broadcast_to(row0, (16,)) + iota
            svec = plsc.load_gather(
                rows_v, [rowids, jnp.full((16,), D - 1, jnp.int32)])
            gvec = jnp.broadcast_to(gbase, (16,)) + iota
            fresh_vec = gvec >= fresh_from
            s_eff = jnp.where(fresh_vec, svec, jnp.float32(NEG))
            thr = plsc.load_gather(
                ts_s, [bv, jnp.full((16,), 2, jnp.int32)])
            npass = plsc.all_reduce_population_count(s_eff > thr)[0]
            b0 = bv[0]
            uniform = jnp.logical_and(b0 == bv[15], gbase >= fresh_from)

            @pl.when(uniform)
            def _():
                def ucol(c4, _):
                    for u in range(4):
                        cbase = pl.multiple_of(c4 * 64 + u * 16, 16)
                        vs = [rows_v[row0 + l, pl.ds(cbase, 16)]
                              for l in range(16)]
                        while len(vs) > 1:
                            vs = [vs[i] + vs[i + 1]
                                  for i in range(0, len(vs), 2)]
                        plsc.addupdate(acc_v.at[b0, pl.ds(cbase, 16)], vs[0])
                    return 0

                lax.fori_loop(0, D // 64, ucol, 0)
                plsc.addupdate(cnt_v.at[b0],
                               jnp.full((16,), 16.0, jnp.float32))

            @pl.when(jnp.logical_not(uniform))
            def _():
                def lane_body(l, _):
                    lv = jnp.broadcast_to(l, (16,))
                    b = bv.at[lv].get(mode="promise_in_bounds")[0]
                    g = gbase + l
                    fresh = g >= fresh_from
                    beff = jnp.where(fresh, b, jnp.int32(B))
                    onev = jnp.broadcast_to(
                        jnp.where(fresh, jnp.float32(1.0), jnp.float32(0.0)),
                        (16,))
                    plsc.addupdate(cnt_v.at[beff], onev)
                    row = row0 + l

                    def fcol(c4, _):
                        for u in range(4):
                            cbase = pl.multiple_of(c4 * 64 + u * 16, 16)
                            v = rows_v[row, pl.ds(cbase, 16)]
                            plsc.addupdate(
                                acc_v.at[beff, pl.ds(cbase, 16)], v)
                        return 0

                    lax.fori_loop(0, D // 64, fcol, 0)
                    return 0

                lax.fori_loop(0, 16, lane_body, 0)

            @pl.when(npass > 0)
            def _():
                for l in range(16):
                    se = s_eff[l]
                    b = bv[l]
                    g = gbase + l
                    sv = ts_s[b, :]
                    iv = ts_i[b, :]
                    n0s, n1s, n2s, n0i, n1i, n2i = _insert3(
                        se, g, sv[0], sv[1], sv[2], iv[0], iv[1], iv[2])
                    ns = jnp.where(iota == 0, n0s,
                                   jnp.where(iota == 1, n1s,
                                             jnp.where(iota == 2, n2s, sv)))
                    ni = jnp.where(iota == 0, n0i,
                                   jnp.where(iota == 1, n1i,
                                             jnp.where(iota == 2, n2i, iv)))
                    ts_s[b, :] = ns
                    ts_i[b, :] = ni

            return 0

        lax.fori_loop(0, S // 16, grp, 0)

    pltpu.async_copy(x_hbm.at[pl.ds(cs_of(0), S)], rows0_v, sem_r0)
    pltpu.async_copy(bat_hbm.at[pl.ds(cs_of(0), S)], idx0_v, sem_i0)
    npairs = (nc + 1) // 2

    def pair_body(p, _):
        j0 = 2 * p
        pltpu.make_async_copy(
            x_hbm.at[pl.ds(cs_of(j0), S)], rows0_v, sem_r0).wait()
        pltpu.make_async_copy(
            bat_hbm.at[pl.ds(cs_of(j0), S)], idx0_v, sem_i0).wait()

        @pl.when(j0 + 1 < nc)
        def _():
            pltpu.async_copy(
                x_hbm.at[pl.ds(cs_of(j0 + 1), S)], rows1_v, sem_r1)
            pltpu.async_copy(
                bat_hbm.at[pl.ds(cs_of(j0 + 1), S)], idx1_v, sem_i1)

        process(rows0_v, idx0_v, j0)

        @pl.when(j0 + 1 < nc)
        def _():
            pltpu.make_async_copy(
                x_hbm.at[pl.ds(cs_of(j0 + 1), S)], rows1_v, sem_r1).wait()
            pltpu.make_async_copy(
                bat_hbm.at[pl.ds(cs_of(j0 + 1), S)], idx1_v, sem_i1).wait()

            @pl.when(j0 + 2 < nc)
            def _():
                pltpu.async_copy(
                    x_hbm.at[pl.ds(cs_of(j0 + 2), S)], rows0_v, sem_r0)
                pltpu.async_copy(
                    bat_hbm.at[pl.ds(cs_of(j0 + 2), S)], idx0_v, sem_i0)

            process(rows1_v, idx1_v, j0 + 1)

        return 0

    lax.fori_loop(0, npairs, pair_body, 0)

    pltpu.sync_copy(acc_v, psum_hbm.at[wid])
    pltpu.sync_copy(cnt_v, pcnt_hbm.at[wid])
    pltpu.sync_copy(ts_s, cands_hbm.at[wid])
    pltpu.sync_copy(ts_i, candi_hbm.at[wid])


@functools.partial(
    pl.kernel,
    out_type=jax.ShapeDtypeStruct((B, 5 * D), jnp.float32),
    mesh=_mesh,
    compiler_params=pltpu.CompilerParams(use_tc_tiling_on_sc=True, needs_layout_passes=False),
    scratch_types=[
        pltpu.VMEM((NW, 1, 16), jnp.float32),
        pltpu.VMEM((NW, 1, 16), jnp.int32),
        pltpu.VMEM((NW, 1, D), jnp.float32),
        pltpu.VMEM((NW, 1, 16), jnp.float32),
        pltpu.VMEM((NW, 1, 16), jnp.float32),
        pltpu.VMEM((NW, 1, 16), jnp.int32),
        pltpu.VMEM((NW, 1, D), jnp.float32),
        pltpu.VMEM((NW, 1, 16), jnp.float32),
        pltpu.VMEM((16,), jnp.int32),
        pltpu.VMEM((16, D), jnp.float32),
        pltpu.VMEM((5 * D,), jnp.float32),
        pltpu.VMEM((5 * D,), jnp.float32),
        pltpu.SemaphoreType.DMA,
        pltpu.SemaphoreType.DMA,
        pltpu.SemaphoreType.DMA,
        pltpu.SemaphoreType.DMA,
    ],
)
def _k2(x_hbm, psum_hbm, pcnt_hbm, cands_hbm, candi_hbm, out_hbm,
        cs0, ci0, ps0, pc0, cs1, ci1, ps1, pc1, gi_v, grows_v,
        orow0, orow1, sem_in0, sem_in1, sem_g, sem_out):
    cid = lax.axis_index("c")
    sid = lax.axis_index("s")
    wid = cid * 16 + sid
    iota = lax.iota(jnp.int32, 16)
    zeros16 = jnp.zeros((16,), jnp.float32)

    def in_copies(seg, cs_v, ci_v, psv, pcv, sem):
        return (
            pltpu.make_async_copy(
                cands_hbm.at[:, pl.ds(seg, 1), :], cs_v, sem),
            pltpu.make_async_copy(
                candi_hbm.at[:, pl.ds(seg, 1), :], ci_v, sem),
            pltpu.make_async_copy(
                psum_hbm.at[:, pl.ds(seg, 1), :], psv, sem),
            pltpu.make_async_copy(
                pcnt_hbm.at[:, pl.ds(seg, 1), :], pcv, sem),
        )

    def do_seg(seg, cs_v, ci_v, psv, pcv, orow_v, sem):
        for c in in_copies(seg, cs_v, ci_v, psv, pcv, sem):
            c.wait()

        def m_body(t, carry):
            t0s, t1s, t2s, t0i, t1i, t2i = carry
            csv = cs_v[t, 0, :]
            civ = ci_v[t, 0, :]
            for k in range(3):
                t0s, t1s, t2s, t0i, t1i, t2i = _insert3(
                    csv[k], civ[k], t0s, t1s, t2s, t0i, t1i, t2i)
            return (t0s, t1s, t2s, t0i, t1i, t2i)

        z = jnp.int32(0)
        ng = jnp.float32(NEG)
        t0s, t1s, t2s, t0i, t1i, t2i = lax.fori_loop(
            0, NW, m_body, (ng, ng, ng, z, z, z))

        def cnt_body(t, a):
            return a + pcv[t, 0, :]

        cnt = lax.fori_loop(0, NW, cnt_body, zeros16)
        cntc = jnp.maximum(cnt, jnp.float32(1.0))
        one = jnp.full((16,), 1.0, jnp.float32)
        v0 = jnp.where(cnt > 0.5, one, zeros16)
        v1 = jnp.where(cnt > 1.5, one, zeros16)
        v2 = jnp.where(cnt > 2.5, one, zeros16)

        gi_v[...] = jnp.where(iota == 0, t0i,
                              jnp.where(iota == 1, t1i,
                                        jnp.where(iota == 2, t2i, z)))
        pltpu.async_copy(x_hbm.at[gi_v], grows_v, sem_g).wait()

        def col_body(c4, _):
            bases = [pl.multiple_of(c4 * 64 + u * 16, 16) for u in range(4)]

            def s_body(t, accs):
                return tuple(a + psv[t, 0, pl.ds(bases[u], 16)]
                             for u, a in enumerate(accs))

            svs = lax.fori_loop(0, NW, s_body, (zeros16,) * 4)
            for u in range(4):
                base = bases[u]
                sl = pl.ds(base, 16)
                sv = svs[u]
                orow_v[pl.ds(base, 16)] = sv / cntc
                orow_v[pl.ds(D + base, 16)] = sv
                orow_v[pl.ds(2 * D + base, 16)] = grows_v[0, sl] * v0
                orow_v[pl.ds(3 * D + base, 16)] = grows_v[1, sl] * v1
                orow_v[pl.ds(4 * D + base, 16)] = grows_v[2, sl] * v2
            return 0

        lax.fori_loop(0, D // 64, col_body, 0)
        pltpu.async_copy(orow_v, out_hbm.at[seg], sem_out)

    seg_a = wid * 2
    seg_b = seg_a + 1
    for c in in_copies(seg_a, cs0, ci0, ps0, pc0, sem_in0):
        c.start()
    for c in in_copies(seg_b, cs1, ci1, ps1, pc1, sem_in1):
        c.start()
    do_seg(seg_a, cs0, ci0, ps0, pc0, orow0, sem_in0)
    do_seg(seg_b, cs1, ci1, ps1, pc1, orow1, sem_in1)
    pltpu.make_async_copy(orow0, out_hbm.at[seg_a], sem_out).wait()
    pltpu.make_async_copy(orow1, out_hbm.at[seg_b], sem_out).wait()


def kernel(x, batch):
    bat = batch.astype(jnp.int32)
    psum, pcnt, cs, ci = _k1(x, bat)
    return _k2(x, psum, pcnt, cs, ci)


# speedup vs baseline: 12.0473x; 1.0978x over previous
import functools

import jax
import jax.numpy as jnp
from jax import lax
from jax.experimental import pallas as pl
from jax.experimental.pallas import tpu as pltpu
from jax.experimental.pallas import tpu_sc as plsc

N = 50000
D = 512
B = 64
NW = 32
PER = 1568
BBLK = 1024
NB = -(-N // BBLK)
NEG = -3.0e38

_mesh = plsc.VectorSubcoreMesh(core_axis_name="c", subcore_axis_name="s")
_sc_params = pltpu.CompilerParams(use_tc_tiling_on_sc=True,
                                  needs_layout_passes=False)


def _insert3(cs, ci, t0s, t1s, t2s, t0i, t1i, t2i):
    gt0 = cs > t0s
    gt1 = cs > t1s
    gt2 = cs > t2s
    n0s = jnp.where(gt0, cs, t0s)
    n0i = jnp.where(gt0, ci, t0i)
    n1s = jnp.where(gt0, t0s, jnp.where(gt1, cs, t1s))
    n1i = jnp.where(gt0, t0i, jnp.where(gt1, ci, t1i))
    n2s = jnp.where(gt1, t1s, jnp.where(gt2, cs, t2s))
    n2i = jnp.where(gt1, t1i, jnp.where(gt2, ci, t2i))
    return n0s, n1s, n2s, n0i, n1i, n2i


def _ktc_body(x_ref, b_ref, psum_ref, pcnt_ref, sc_ref):
    i = pl.program_id(0)
    xb = x_ref[...]
    bb = b_ref[...]
    rowmask = (lax.broadcasted_iota(jnp.int32, (BBLK, D), 0)
               + i * BBLK) < N
    xb = jnp.where(rowmask, xb, jnp.float32(0.0))
    seg = lax.broadcasted_iota(jnp.int32, (B, BBLK), 0)
    gcol = lax.broadcasted_iota(jnp.int32, (B, BBLK), 1) + i * BBLK
    onehot = jnp.where((seg == bb[None, :]) & (gcol < N), jnp.float32(1.0),
                       jnp.float32(0.0))
    ps = jnp.dot(onehot, xb, preferred_element_type=jnp.float32)
    cnt = jnp.broadcast_to(jnp.sum(onehot, axis=1, keepdims=True), (B, 128))

    @pl.when(i == 0)
    def _():
        psum_ref[...] = ps
        pcnt_ref[...] = cnt

    @pl.when(i > 0)
    def _():
        psum_ref[...] = psum_ref[...] + ps
        pcnt_ref[...] = pcnt_ref[...] + cnt

    sc_ref[...] = xb[:, D - 1]


_ktc = pl.pallas_call(
    _ktc_body,
    grid=(NB,),
    in_specs=[
        pl.BlockSpec((BBLK, D), lambda i: (i, 0)),
        pl.BlockSpec((BBLK,), lambda i: (i,)),
    ],
    out_specs=[
        pl.BlockSpec((B, D), lambda i: (0, 0)),
        pl.BlockSpec((B, 128), lambda i: (0, 0)),
        pl.BlockSpec((BBLK,), lambda i: (i,)),
    ],
    out_shape=[
        jax.ShapeDtypeStruct((B, D), jnp.float32),
        jax.ShapeDtypeStruct((B, 128), jnp.float32),
        jax.ShapeDtypeStruct((N,), jnp.float32),
    ],
)


@functools.partial(
    pl.kernel,
    out_type=(
        jax.ShapeDtypeStruct((NW, B, 16), jnp.float32),
        jax.ShapeDtypeStruct((NW, B, 16), jnp.int32),
    ),
    mesh=_mesh,
    compiler_params=_sc_params,
    scratch_types=[
        pltpu.VMEM((PER,), jnp.float32),
        pltpu.VMEM((PER,), jnp.int32),
        pltpu.VMEM((B, 16), jnp.float32),
        pltpu.VMEM((B, 16), jnp.int32),
    ],
)
def _ksc1(sc_hbm, bat_hbm, cands_hbm, candi_hbm, sc_v, idx_v, ts_s, ts_i):
    cid = lax.axis_index("c")
    sid = lax.axis_index("s")
    wid = cid * 16 + sid
    start = wid * PER
    sstart = jnp.minimum(start, N - PER)
    iota = lax.iota(jnp.int32, 16)

    def init_body(r, _):
        ts_s[r, :] = jnp.full((16,), NEG, jnp.float32)
        ts_i[r, :] = jnp.zeros((16,), jnp.int32)
        return 0

    lax.fori_loop(0, B, init_body, 0)
    pltpu.sync_copy(sc_hbm.at[pl.ds(sstart, PER)], sc_v)
    pltpu.sync_copy(bat_hbm.at[pl.ds(sstart, PER)], idx_v)

    def grp(kk, _):
        goff = pl.multiple_of(kk * 16, 16)
        bv = idx_v[pl.ds(goff, 16)]
        svec = sc_v[pl.ds(goff, 16)]
        gbase = sstart + kk * 16
        gvec = jnp.broadcast_to(gbase, (16,)) + iota
        s_eff = jnp.where(gvec >= start, svec, jnp.float32(NEG))
        thr = plsc.load_gather(ts_s, [bv, jnp.full((16,), 2, jnp.int32)])
        npass = plsc.all_reduce_population_count(s_eff > thr)[0]

        @pl.when(npass > 0)
        def _():
            for l in range(16):
                se = s_eff[l]
                b = bv[l]
                g = gbase + l
                sv = ts_s[b, :]
                iv = ts_i[b, :]
                n0s, n1s, n2s, n0i, n1i, n2i = _insert3(
                    se, g, sv[0], sv[1], sv[2], iv[0], iv[1], iv[2])
                ns = jnp.where(iota == 0, n0s,
                               jnp.where(iota == 1, n1s,
                                         jnp.where(iota == 2, n2s, sv)))
                ni = jnp.where(iota == 0, n0i,
                               jnp.where(iota == 1, n1i,
                                         jnp.where(iota == 2, n2i, iv)))
                ts_s[b, :] = ns
                ts_i[b, :] = ni

        return 0

    lax.fori_loop(0, PER // 16, grp, 0)
    pltpu.sync_copy(ts_s, cands_hbm.at[wid])
    pltpu.sync_copy(ts_i, candi_hbm.at[wid])


@functools.partial(
    pl.kernel,
    out_type=jax.ShapeDtypeStruct((B, 5 * D), jnp.float32),
    mesh=_mesh,
    compiler_params=_sc_params,
    scratch_types=[
        pltpu.VMEM((NW, 1, 16), jnp.float32),
        pltpu.VMEM((NW, 1, 16), jnp.int32),
        pltpu.VMEM((1, D), jnp.float32),
        pltpu.VMEM((1, 128), jnp.float32),
        pltpu.VMEM((NW, 1, 16), jnp.float32),
        pltpu.VMEM((NW, 1, 16), jnp.int32),
        pltpu.VMEM((1, D), jnp.float32),
        pltpu.VMEM((1, 128), jnp.float32),
        pltpu.VMEM((16,), jnp.int32),
        pltpu.VMEM((16, D), jnp.float32),
        pltpu.VMEM((5 * D,), jnp.float32),
        pltpu.VMEM((5 * D,), jnp.float32),
        pltpu.SemaphoreType.DMA,
        pltpu.SemaphoreType.DMA,
        pltpu.SemaphoreType.DMA,
        pltpu.SemaphoreType.DMA,
    ],
)
def _ksc2(x_hbm, psum_hbm, pcnt_hbm, cands_hbm, candi_hbm, out_hbm,
          cs0, ci0, ps0, pc0, cs1, ci1, ps1, pc1, gi_v, grows_v,
          orow0, orow1, sem_in0, sem_in1, sem_g, sem_out):
    cid = lax.axis_index("c")
    sid = lax.axis_index("s")
    wid = cid * 16 + sid
    iota = lax.iota(jnp.int32, 16)
    zeros16 = jnp.zeros((16,), jnp.float32)

    def in_copies(seg, cs_v, ci_v, psv, pcv, sem):
        return (
            pltpu.make_async_copy(
                cands_hbm.at[:, pl.ds(seg, 1), :], cs_v, sem),
            pltpu.make_async_copy(
                candi_hbm.at[:, pl.ds(seg, 1), :], ci_v, sem),
            pltpu.make_async_copy(
                psum_hbm.at[pl.ds(seg, 1), :], psv, sem),
            pltpu.make_async_copy(
                pcnt_hbm.at[pl.ds(seg, 1), :], pcv, sem),
        )

    def do_seg(seg, cs_v, ci_v, psv, pcv, orow_v, sem):
        for c in in_copies(seg, cs_v, ci_v, psv, pcv, sem):
            c.wait()

        def m_body(t, carry):
            t0s, t1s, t2s, t0i, t1i, t2i = carry
            csv = cs_v[t, 0, :]
            civ = ci_v[t, 0, :]
            for k in range(3):
                t0s, t1s, t2s, t0i, t1i, t2i = _insert3(
                    csv[k], civ[k], t0s, t1s, t2s, t0i, t1i, t2i)
            return (t0s, t1s, t2s, t0i, t1i, t2i)

        z = jnp.int32(0)
        ng = jnp.float32(NEG)
        t0s, t1s, t2s, t0i, t1i, t2i = lax.fori_loop(
            0, NW, m_body, (ng, ng, ng, z, z, z))

        cnt = pcv[0, pl.ds(0, 16)]
        cntc = jnp.maximum(cnt, jnp.float32(1.0))
        one = jnp.full((16,), 1.0, jnp.float32)
        v0 = jnp.where(cnt > 0.5, one, zeros16)
        v1 = jnp.where(cnt > 1.5, one, zeros16)
        v2 = jnp.where(cnt > 2.5, one, zeros16)

        gi_v[...] = jnp.where(iota == 0, t0i,
                              jnp.where(iota == 1, t1i,
                                        jnp.where(iota == 2, t2i, z)))
        pltpu.async_copy(x_hbm.at[gi_v], grows_v, sem_g).wait()

        def col_body(c4, _):
            bases = [pl.multiple_of(c4 * 64 + u * 16, 16) for u in range(4)]
            for u in range(4):
                base = bases[u]
                sl = pl.ds(base, 16)
                sv = psv[0, sl]
                orow_v[pl.ds(base, 16)] = sv / cntc
                orow_v[pl.ds(D + base, 16)] = sv
                orow_v[pl.ds(2 * D + base, 16)] = grows_v[0, sl] * v0
                orow_v[pl.ds(3 * D + base, 16)] = grows_v[1, sl] * v1
                orow_v[pl.ds(4 * D + base, 16)] = grows_v[2, sl] * v2
            return 0

        lax.fori_loop(0, D // 64, col_body, 0)
        pltpu.async_copy(orow_v, out_hbm.at[seg], sem_out)

    seg_a = wid * 2
    seg_b = seg_a + 1
    for c in in_copies(seg_a, cs0, ci0, ps0, pc0, sem_in0):
        c.start()
    for c in in_copies(seg_b, cs1, ci1, ps1, pc1, sem_in1):
        c.start()
    do_seg(seg_a, cs0, ci0, ps0, pc0, orow0, sem_in0)
    do_seg(seg_b, cs1, ci1, ps1, pc1, orow1, sem_in1)
    pltpu.make_async_copy(orow0, out_hbm.at[seg_a], sem_out).wait()
    pltpu.make_async_copy(orow1, out_hbm.at[seg_b], sem_out).wait()


def kernel(x, batch):
    bat = batch.astype(jnp.int32)
    psum, pcnt, scores = _ktc(x, bat)
    cs, ci = _ksc1(scores, bat)
    return _ksc2(x, psum, pcnt, cs, ci)
